# Initial kernel scaffold; baseline (speedup 1.0000x reference)
import jax
import jax.numpy as jnp
from jax.experimental import pallas as pl


def kernel(h_n0, h_n1, h_n2, edge_index, idx_n0_n0, idx_n0_n1, idx_n0_n2, idx_n1_n0, idx_n1_n1, idx_n1_n2, idx_n2_n0, idx_n2_n1, idx_n2_n2, W_proj_n0, W_ctx_n0, W_hid_n0, W_out_n0, W_proj_n1, W_ctx_n1, W_hid_n1, W_out_n1, W_proj_n2, W_ctx_n2, W_hid_n2, W_out_n2, W_gnn, b_gnn):
    raise NotImplementedError("write your pallas kernel here")


# speedup vs baseline: 3.9307x; 3.9307x over previous
import functools

import jax
import jax.numpy as jnp
from jax import lax
from jax.experimental import pallas as pl
from jax.experimental.pallas import tpu as pltpu
from jax.experimental.pallas import tpu_sc as plsc

N0, N1, N2 = 4000, 4000, 2000
N = N0 + N1 + N2
E = 320000
D_IN, D_PROJ, D_EMD = 128, 64, 64
D_CTX, D_HID = 16, 16
S = 16384

NC, NS = 2, 16
NW = NC * NS
EDGE_K = 128
E_PAD = 327680
CHUNKS = E_PAD // (NW * EDGE_K)
N_ACC = 10240
ROWS_PER_TILE = N_ACC // NS

G_TOTAL = 9 * S
G_CHUNKS = G_TOTAL // (NW * 128)
G_PER_TILE = G_CHUNKS * 128


def _proj_body(x_ref, w_ref, o_ref):
    o_ref[...] = jnp.dot(x_ref[...], w_ref[0],
                         preferred_element_type=jnp.float32)


def _proj(x, wp):
    return pl.pallas_call(
        _proj_body,
        grid=(5,),
        in_specs=[
            pl.BlockSpec((2000, D_IN), lambda i: (i, 0)),
            pl.BlockSpec((1, D_IN, D_PROJ), lambda i: (i // 2, 0, 0)),
        ],
        out_specs=pl.BlockSpec((2000, D_PROJ), lambda i: (i, 0)),
        out_shape=jax.ShapeDtypeStruct((N, D_PROJ), jnp.float32),
    )(x, wp)


def _segsum_body(h_hbm, src_hbm, dst_hbm, zinit_hbm, out_hbm,
                 src_v, dst_v, buf, acc_sh, sem):
    cid = lax.axis_index("c")
    sid = lax.axis_index("s")
    wid = cid * NS + sid
    pltpu.sync_copy(src_hbm.at[wid], src_v)
    pltpu.sync_copy(dst_hbm.at[wid], dst_v)
    r0 = sid * ROWS_PER_TILE
    pltpu.sync_copy(zinit_hbm.at[pl.ds(r0, ROWS_PER_TILE)],
                    acc_sh.at[pl.ds(r0, ROWS_PER_TILE)])
    plsc.subcore_barrier()

    def body(j, _):
        pltpu.async_copy(h_hbm.at[src_v.at[j]], buf, sem).wait()
        pltpu.sync_copy(buf, acc_sh.at[dst_v.at[j]], add=True)
        return _

    lax.fori_loop(0, CHUNKS, body, None)
    plsc.subcore_barrier()
    pltpu.sync_copy(acc_sh.at[pl.ds(r0, ROWS_PER_TILE)],
                    out_hbm.at[pl.ds(cid * N_ACC + r0, ROWS_PER_TILE)])


def _segsum(h_homo, src_p, dst_p, zinit):
    mesh = plsc.VectorSubcoreMesh(core_axis_name="c", subcore_axis_name="s")
    k = functools.partial(
        pl.kernel,
        mesh=mesh,
        out_type=jax.ShapeDtypeStruct((NC * N_ACC, D_PROJ), jnp.float32),
        scratch_types=[
            pltpu.VMEM((CHUNKS, EDGE_K), jnp.int32),
            pltpu.VMEM((CHUNKS, EDGE_K), jnp.int32),
            pltpu.VMEM((EDGE_K, D_PROJ), jnp.float32),
            pltpu.VMEM_SHARED((N_ACC, D_PROJ), jnp.float32),
            pltpu.SemaphoreType.DMA,
        ],
        compiler_params=pltpu.CompilerParams(use_tc_tiling_on_sc=False),
    )(_segsum_body)
    return k(h_homo, src_p, dst_p, zinit)


def _gnn_body(a0_ref, a1_ref, w_ref, b_ref, o_ref):
    h = jnp.dot(a0_ref[...] + a1_ref[...], w_ref[...],
                preferred_element_type=jnp.float32) + b_ref[...]
    nrm = jnp.sqrt(jnp.sum(h * h, axis=1, keepdims=True))
    o_ref[...] = h / jnp.maximum(nrm, 1e-12)


def _gnn(a0, a1, w_gnn, b_gnn):
    return pl.pallas_call(
        _gnn_body,
        grid=(5,),
        in_specs=[
            pl.BlockSpec((2000, D_PROJ), lambda i: (i, 0)),
            pl.BlockSpec((2000, D_PROJ), lambda i: (i, 0)),
            pl.BlockSpec((D_PROJ, D_EMD), lambda i: (0, 0)),
            pl.BlockSpec((1, D_EMD), lambda i: (0, 0)),
        ],
        out_specs=pl.BlockSpec((2000, D_EMD), lambda i: (i, 0)),
        out_shape=jax.ShapeDtypeStruct((N, D_EMD), jnp.float32),
    )(a0, a1, w_gnn, b_gnn)


def _tables_body(e_ref, wh_ref, wc_ref, o_ref):
    t = pl.program_id(0)
    j = pl.program_id(1)
    typ = j // 2
    pos = jnp.where(typ < t, typ, typ - 1)
    m_same = wh_ref[0, :D_EMD, :]
    p0 = wh_ref[0, D_EMD:D_EMD + D_CTX, :]
    p1 = wh_ref[0, D_EMD + D_CTX:D_EMD + 2 * D_CTX, :]
    part = jnp.where(pos == 0, p0, p1)
    m_diff = jnp.dot(wc_ref[0], part, preferred_element_type=jnp.float32)
    m = jnp.where(typ == t, m_same, m_diff)
    o_ref[0] = jnp.dot(e_ref[...], m, preferred_element_type=jnp.float32)


def _tables(emd, wh_all, wc_all):
    return pl.pallas_call(
        _tables_body,
        grid=(3, 5),
        in_specs=[
            pl.BlockSpec((2000, D_EMD), lambda t, j: (j, 0)),
            pl.BlockSpec((1, D_EMD + 2 * D_CTX, D_HID), lambda t, j: (t, 0, 0)),
            pl.BlockSpec((1, D_EMD, D_HID), lambda t, j: (j // 2, 0, 0)),
        ],
        out_specs=pl.BlockSpec((1, 2000, D_HID), lambda t, j: (t, j, 0)),
        out_shape=jax.ShapeDtypeStruct((3, N, D_HID), jnp.float32),
    )(emd, wh_all, wc_all)


def _clsgather_body(tab_hbm, idx_hbm, out_hbm, idx_v, out_v, sem):
    cid = lax.axis_index("c")
    sid = lax.axis_index("s")
    wid = cid * NS + sid
    pltpu.sync_copy(idx_hbm.at[wid], idx_v)

    def body(j, _):
        pltpu.async_copy(tab_hbm.at[idx_v.at[j]],
                         out_v.at[pl.ds(j * 128, 128)], sem).wait()
        return _

    lax.fori_loop(0, G_CHUNKS, body, None)
    pltpu.sync_copy(out_v, out_hbm.at[pl.ds(wid * G_PER_TILE, G_PER_TILE)])


def _clsgather(tab_flat, g_idx):
    mesh = plsc.VectorSubcoreMesh(core_axis_name="c", subcore_axis_name="s")
    k = functools.partial(
        pl.kernel,
        mesh=mesh,
        out_type=jax.ShapeDtypeStruct((G_TOTAL, D_HID), jnp.float32),
        scratch_types=[
            pltpu.VMEM((G_CHUNKS, 128), jnp.int32),
            pltpu.VMEM((G_PER_TILE, D_HID), jnp.float32),
            pltpu.SemaphoreType.DMA,
        ],
        compiler_params=pltpu.CompilerParams(use_tc_tiling_on_sc=False),
    )(_clsgather_body)
    return k(tab_flat, g_idx)


def _cls_body(a_ref, b_ref, c_ref, wo_ref, o_ref):
    hid = jax.nn.relu(a_ref[...] + b_ref[...] + c_ref[...])
    p = jnp.dot(hid, wo_ref[0], preferred_element_type=jnp.float32)
    o_ref[...] = jax.nn.sigmoid(p)


def _cls(ag, bg, cg, wo_all):
    return pl.pallas_call(
        _cls_body,
        grid=(48,),
        in_specs=[
            pl.BlockSpec((1024, D_HID), lambda i: (i, 0)),
            pl.BlockSpec((1024, D_HID), lambda i: (i, 0)),
            pl.BlockSpec((1024, D_HID), lambda i: (i, 0)),
            pl.BlockSpec((1, D_HID, 1), lambda i: (i // 16, 0, 0)),
        ],
        out_specs=pl.BlockSpec((1024, 1), lambda i: (i, 0)),
        out_shape=jax.ShapeDtypeStruct((3 * S, 1), jnp.float32),
    )(ag, bg, cg, wo_all)


def kernel(h_n0, h_n1, h_n2, edge_index,
           idx_n0_n0, idx_n0_n1, idx_n0_n2,
           idx_n1_n0, idx_n1_n1, idx_n1_n2,
           idx_n2_n0, idx_n2_n1, idx_n2_n2,
           W_proj_n0, W_ctx_n0, W_hid_n0, W_out_n0,
           W_proj_n1, W_ctx_n1, W_hid_n1, W_out_n1,
           W_proj_n2, W_ctx_n2, W_hid_n2, W_out_n2,
           W_gnn, b_gnn):
    ei = edge_index.astype(jnp.int32)
    pad = E_PAD - E
    src_p = jnp.concatenate(
        [ei[0], jnp.zeros((pad,), jnp.int32)]).reshape(NW, CHUNKS, EDGE_K)
    dst_p = jnp.concatenate(
        [ei[1], jnp.full((pad,), N_ACC - 1, jnp.int32)]).reshape(NW, CHUNKS, EDGE_K)

    x = jnp.concatenate([h_n0, h_n1, h_n2], axis=0)
    wp = jnp.stack([W_proj_n0, W_proj_n1, W_proj_n2])
    h_homo = _proj(x, wp)
    zinit = jnp.zeros((N_ACC, D_PROJ), jnp.float32)
    acc2 = _segsum(h_homo, src_p, dst_p, zinit)

    h = _gnn(acc2[:N], acc2[N_ACC:N_ACC + N], W_gnn, b_gnn.reshape(1, D_EMD))

    wh_all = jnp.stack([W_hid_n0, W_hid_n1, W_hid_n2])
    wc_all = jnp.stack([W_ctx_n0, W_ctx_n1, W_ctx_n2])
    tab = _tables(h, wh_all, wc_all).reshape(3 * N, D_HID)

    off = (0, N0, N0 + N1)
    a_idx = jnp.concatenate([
        0 * N + off[0] + idx_n0_n0,
        1 * N + off[1] + idx_n1_n1,
        2 * N + off[2] + idx_n2_n2])
    b_idx = jnp.concatenate([
        0 * N + off[1] + idx_n0_n1,
        1 * N + off[0] + idx_n1_n0,
        2 * N + off[0] + idx_n2_n0])
    c_idx = jnp.concatenate([
        0 * N + off[2] + idx_n0_n2,
        1 * N + off[2] + idx_n1_n2,
        2 * N + off[1] + idx_n2_n1])
    g_idx = jnp.concatenate([a_idx, b_idx, c_idx]).astype(jnp.int32)
    g_idx = g_idx.reshape(NW, G_CHUNKS, 128)

    gout = _clsgather(tab, g_idx)

    wo_all = jnp.stack([W_out_n0, W_out_n1, W_out_n2])
    p = _cls(gout[:3 * S], gout[3 * S:6 * S], gout[6 * S:], wo_all)
    return (h, p.reshape(-1))
